# baseline (speedup 1.0000x reference)
import functools

import jax
import jax.numpy as jnp
from jax import lax
from jax.experimental import pallas as pl
from jax.experimental.pallas import tpu as pltpu
from jax.experimental.pallas import tpu_sc as plsc

N_ITEMS = 100000
BATCH_N = 1024
SEQ_N = 50
HID_N = 64
SPAD = 64
BB = 256

_NC = 2
_NPASS = 50
_GPP = 125
_SLAB = _GPP * 8192
_FLAT = N_ITEMS * BATCH_N


def _probs_body(am_ref, lm_ref, urT_ref, wrT_ref, vr_ref, out_ref, sc_ref):
    lm = jnp.dot(lm_ref[...], wrT_ref[...], preferred_element_type=jnp.float32)
    vr = vr_ref[...]
    for s in range(SEQ_N):
        xs = am_ref[:, s, :]
        zs = jnp.dot(xs, urT_ref[...], preferred_element_type=jnp.float32) + lm
        ts = jnp.tanh(zs)
        sc_ref[:, s:s + 1] = jnp.sum(ts * vr, axis=1, keepdims=True)
    scores = sc_ref[...]
    colid = lax.broadcasted_iota(jnp.int32, (BB, SPAD), 1)
    valid = colid < SEQ_N
    smax = jnp.max(jnp.where(valid, scores, -jnp.inf), axis=1, keepdims=True)
    e = jnp.where(valid, jnp.exp(scores - smax), 0.0)
    out_ref[...] = e / jnp.sum(e, axis=1, keepdims=True)


def _probs_call(all_memory, last_memory, urT, wrT, vr):
    return pl.pallas_call(
        _probs_body,
        grid=(BATCH_N // BB,),
        in_specs=[
            pl.BlockSpec((BB, SEQ_N, HID_N), lambda i: (i, 0, 0)),
            pl.BlockSpec((BB, HID_N), lambda i: (i, 0)),
            pl.BlockSpec((HID_N, HID_N), lambda i: (0, 0)),
            pl.BlockSpec((HID_N, HID_N), lambda i: (0, 0)),
            pl.BlockSpec((1, HID_N), lambda i: (0, 0)),
        ],
        out_specs=pl.BlockSpec((BB, SPAD), lambda i: (i, 0)),
        out_shape=jax.ShapeDtypeStruct((BATCH_N, SPAD), jnp.float32),
        scratch_shapes=[pltpu.VMEM((BB, SPAD), jnp.float32)],
    )(all_memory, last_memory, urT, wrT, vr)


@functools.cache
def _get_scatter_kernel():
    mesh = plsc.VectorSubcoreMesh(core_axis_name="c", subcore_axis_name="s")

    @functools.partial(
        pl.kernel,
        mesh=mesh,
        compiler_params=pltpu.CompilerParams(needs_layout_passes=False),
        out_type=jax.ShapeDtypeStruct((_FLAT,), jnp.float32),
        scratch_types=[
            pltpu.VMEM_SHARED((_SLAB,), jnp.float32),
            pltpu.VMEM((64, SPAD), jnp.int32),
            pltpu.VMEM((64, SPAD), jnp.float32),
            pltpu.VMEM((64, SPAD), jnp.int32),
            pltpu.VMEM((64, SPAD), jnp.int32),
            pltpu.VMEM((4352,), jnp.int32),
            pltpu.VMEM((4352,), jnp.float32),
            pltpu.VMEM((2, 32, 128), jnp.int32),
            pltpu.VMEM((2, 32, 128), jnp.float32),
            pltpu.VMEM((8000,), jnp.float32),
            pltpu.VMEM((128,), jnp.float32),
            pltpu.SemaphoreType.DMA,
        ],
    )
    def _scatter_kernel(seq_hbm, probs_hbm, out_hbm, slab, idxbuf, updbuf,
                        gidbuf, posbuf, pos1d, val1d, pos2d, val2d,
                        zbuf, zero128, dsem):
        cid = lax.axis_index("c")
        sid = lax.axis_index("s")
        z16 = jnp.zeros((16,), jnp.float32)
        z16i = jnp.zeros((16,), jnp.int32)

        pltpu.sync_copy(seq_hbm.at[pl.ds(sid * 64, 64)], idxbuf)
        pltpu.sync_copy(probs_hbm.at[pl.ds(sid * 64, 64)], updbuf)

        def _z(i, c):
            zbuf[pl.ds(i * 16, 16)] = z16
            return c
        lax.fori_loop(0, 500, _z, 0)

        def _z2(i, c):
            zero128[pl.ds(i * 16, 16)] = z16
            return c
        lax.fori_loop(0, 8, _z2, 0)

        def _zs(i, c):
            pltpu.sync_copy(zbuf, slab.at[pl.ds(sid * 64000 + i * 8000, 8000)])
            return c
        lax.fori_loop(0, 8, _zs, 0)

        def _pre(r, c):
            b = sid * 64 + r
            b_off = (b >> 7) * 1024 + (b & 127)
            for j in range(4):
                sl = pl.ds(j * 16, 16)
                idx = idxbuf[r, sl]
                g = idx >> 3
                gidbuf[r, sl] = g
                posbuf[r, sl] = (g << 13) + ((idx & 7) << 7) + b_off
            return c
        lax.fori_loop(0, 64, _pre, 0)

        def _scan(p_global, parity):
            gb = p_global * _GPP
            base = gb * 8192

            def _srow(r, cnt):
                for j in range(4):
                    sl = pl.ds(j * 16, 16)
                    g = gidbuf[r, sl]
                    m = (g >= gb) & (g < gb + _GPP)
                    plsc.store_compressed(pos1d.at[pl.ds(cnt, 16)],
                                          posbuf[r, sl] - base, mask=m)
                    plsc.store_compressed(val1d.at[pl.ds(cnt, 16)],
                                          updbuf[r, sl], mask=m)
                    cnt = cnt + plsc.all_reduce_population_count(m)[0]
                return cnt

            cnt = lax.fori_loop(0, 64, _srow, jnp.int32(0))
            t = (cnt + 127) >> 7
            pos1d[pl.ds(cnt, 16)] = z16i
            val1d[pl.ds(cnt, 16)] = z16
            cnt2 = (cnt + 15) >> 4

            def _zt(k, c):
                pos1d[pl.ds(k * 16, 16)] = z16i
                val1d[pl.ds(k * 16, 16)] = z16
                return c
            lax.fori_loop(cnt2, t * 16, _zt, 0)

            def _cp(i, c):
                for k in range(8):
                    pos2d[parity, i, pl.ds(k * 16, 16)] = \
                        pos1d[pl.ds(i * 128 + k * 16, 16)]
                    val2d[parity, i, pl.ds(k * 16, 16)] = \
                        val1d[pl.ds(i * 128 + k * 16, 16)]
                return c
            lax.fori_loop(0, t, _cp, 0)
            return t

        gstart = jnp.minimum(sid * 8, _GPP - 8)

        t0 = _scan(cid * _NPASS, jnp.int32(0))

        def _pass(p, t_cur):
            p_global = cid * _NPASS + p
            parity = p & 1

            def _sa(i, c):
                pltpu.sync_copy(val2d.at[parity, i],
                                slab.at[pos2d.at[parity, i]], add=True)
                return c
            lax.fori_loop(0, t_cur, _sa, 0)
            plsc.subcore_barrier()

            gb = p_global * _GPP
            h = pltpu.async_copy(
                slab.at[pl.ds(gstart * 8192, 8 * 8192)],
                out_hbm.at[pl.ds((gb + gstart) * 8192, 8 * 8192)],
                dsem)
            t_next = _scan(p_global + 1, 1 - parity)
            h.wait()
            plsc.subcore_barrier()

            def _us(i, c):
                pltpu.sync_copy(zero128, slab.at[pos2d.at[parity, i]])
                return c
            lax.fori_loop(0, t_cur, _us, 0)
            plsc.subcore_barrier()
            return t_next

        lax.fori_loop(0, _NPASS, _pass, t0)

    return _scatter_kernel


def kernel(all_memory, last_memory, seq_item, Ur_w, Wr_w, Vr_w, Vr_b):
    del Vr_b
    probs = _probs_call(all_memory, last_memory, Ur_w.T, Wr_w.T, Vr_w)
    seq_pad = jnp.pad(seq_item, ((0, 0), (0, SPAD - SEQ_N)))
    flat = _get_scatter_kernel()(seq_pad, probs)
    out = flat.reshape(N_ITEMS // 8, 8, 8, 128).transpose((0, 2, 1, 3))
    return out.reshape(N_ITEMS, BATCH_N).T


# speedup vs baseline: 1.0607x; 1.0607x over previous
import functools

import jax
import jax.numpy as jnp
from jax import lax
from jax.experimental import pallas as pl
from jax.experimental.pallas import tpu as pltpu
from jax.experimental.pallas import tpu_sc as plsc

N_ITEMS = 100000
BATCH_N = 1024
SEQ_N = 50
HID_N = 64
SPAD = 64
BB = 256

_NC = 2
_NPASS = 50
_GPP = 125
_SLAB = _GPP * 8192
_FLAT = N_ITEMS * BATCH_N


def _probs_body(am_ref, lm_ref, urT_ref, wrT_ref, vr_ref, out_ref, sc_ref):
    lm = jnp.dot(lm_ref[...], wrT_ref[...], preferred_element_type=jnp.float32)
    vr = vr_ref[...]
    for s in range(SEQ_N):
        xs = am_ref[:, s, :]
        zs = jnp.dot(xs, urT_ref[...], preferred_element_type=jnp.float32) + lm
        ts = jnp.tanh(zs)
        sc_ref[:, s:s + 1] = jnp.sum(ts * vr, axis=1, keepdims=True)
    scores = sc_ref[...]
    colid = lax.broadcasted_iota(jnp.int32, (BB, SPAD), 1)
    valid = colid < SEQ_N
    smax = jnp.max(jnp.where(valid, scores, -jnp.inf), axis=1, keepdims=True)
    e = jnp.where(valid, jnp.exp(scores - smax), 0.0)
    out_ref[...] = e / jnp.sum(e, axis=1, keepdims=True)


def _probs_call(all_memory, last_memory, urT, wrT, vr):
    return pl.pallas_call(
        _probs_body,
        grid=(BATCH_N // BB,),
        in_specs=[
            pl.BlockSpec((BB, SEQ_N, HID_N), lambda i: (i, 0, 0)),
            pl.BlockSpec((BB, HID_N), lambda i: (i, 0)),
            pl.BlockSpec((HID_N, HID_N), lambda i: (0, 0)),
            pl.BlockSpec((HID_N, HID_N), lambda i: (0, 0)),
            pl.BlockSpec((1, HID_N), lambda i: (0, 0)),
        ],
        out_specs=pl.BlockSpec((BB, SPAD), lambda i: (i, 0)),
        out_shape=jax.ShapeDtypeStruct((BATCH_N, SPAD), jnp.float32),
        scratch_shapes=[pltpu.VMEM((BB, SPAD), jnp.float32)],
    )(all_memory, last_memory, urT, wrT, vr)


@functools.cache
def _get_scatter_kernel():
    mesh = plsc.VectorSubcoreMesh(core_axis_name="c", subcore_axis_name="s")

    @functools.partial(
        pl.kernel,
        mesh=mesh,
        compiler_params=pltpu.CompilerParams(needs_layout_passes=False),
        out_type=jax.ShapeDtypeStruct((_FLAT,), jnp.float32),
        scratch_types=[
            pltpu.VMEM_SHARED((_SLAB,), jnp.float32),
            pltpu.VMEM((64, SPAD), jnp.int32),
            pltpu.VMEM((64, SPAD), jnp.float32),
            pltpu.VMEM((64, SPAD), jnp.int32),
            pltpu.VMEM((64, SPAD), jnp.int32),
            pltpu.VMEM((4352,), jnp.int32),
            pltpu.VMEM((4352,), jnp.float32),
            pltpu.VMEM((2, 32, 128), jnp.int32),
            pltpu.VMEM((2, 32, 128), jnp.float32),
            pltpu.VMEM((8000,), jnp.float32),
            pltpu.VMEM((128,), jnp.float32),
            pltpu.SemaphoreType.DMA,
        ],
    )
    def _scatter_kernel(seq_hbm, probs_hbm, out_hbm, slab, idxbuf, updbuf,
                        gidbuf, posbuf, pos1d, val1d, pos2d, val2d,
                        zbuf, zero128, dsem):
        cid = lax.axis_index("c")
        sid = lax.axis_index("s")
        z16 = jnp.zeros((16,), jnp.float32)
        z16i = jnp.zeros((16,), jnp.int32)

        pltpu.sync_copy(seq_hbm.at[pl.ds(sid * 64, 64)], idxbuf)
        pltpu.sync_copy(probs_hbm.at[pl.ds(sid * 64, 64)], updbuf)

        def _z(i, c):
            zbuf[pl.ds(i * 16, 16)] = z16
            return c
        lax.fori_loop(0, 500, _z, 0)

        def _z2(i, c):
            zero128[pl.ds(i * 16, 16)] = z16
            return c
        lax.fori_loop(0, 8, _z2, 0)

        def _zs(i, c):
            pltpu.sync_copy(zbuf, slab.at[pl.ds(sid * 64000 + i * 8000, 8000)])
            return c
        lax.fori_loop(0, 8, _zs, 0)

        def _pre(r, c):
            b = sid * 64 + r
            b_off = (b >> 7) * 1024 + (b & 127)
            for j in range(4):
                sl = pl.ds(j * 16, 16)
                idx = idxbuf[r, sl]
                g = idx >> 3
                gidbuf[r, sl] = g
                posbuf[r, sl] = (g << 13) + ((idx & 7) << 7) + b_off
            return c
        lax.fori_loop(0, 64, _pre, 0)

        def _scan(p_global, parity):
            gb = p_global * _GPP
            base = gb * 8192

            def _srow(r, cnt):
                for j in range(4):
                    sl = pl.ds(j * 16, 16)
                    g = gidbuf[r, sl]
                    m = (g >= gb) & (g < gb + _GPP)
                    plsc.store_compressed(pos1d.at[pl.ds(cnt, 16)],
                                          posbuf[r, sl] - base, mask=m)
                    plsc.store_compressed(val1d.at[pl.ds(cnt, 16)],
                                          updbuf[r, sl], mask=m)
                    cnt = cnt + plsc.all_reduce_population_count(m)[0]
                return cnt

            cnt = lax.fori_loop(0, 64, _srow, jnp.int32(0))
            t = (cnt + 127) >> 7
            b0 = sid * 64
            d16 = z16i + ((b0 >> 7) * 1024 + (b0 & 127))
            pos1d[pl.ds(cnt, 16)] = d16
            val1d[pl.ds(cnt, 16)] = z16
            cnt2 = (cnt + 15) >> 4

            def _zt(k, c):
                pos1d[pl.ds(k * 16, 16)] = d16
                val1d[pl.ds(k * 16, 16)] = z16
                return c
            lax.fori_loop(cnt2, t * 16, _zt, 0)

            def _cp(i, c):
                for k in range(8):
                    pos2d[parity, i, pl.ds(k * 16, 16)] = \
                        pos1d[pl.ds(i * 128 + k * 16, 16)]
                    val2d[parity, i, pl.ds(k * 16, 16)] = \
                        val1d[pl.ds(i * 128 + k * 16, 16)]
                return c
            lax.fori_loop(0, t, _cp, 0)
            return t

        gstart = jnp.minimum(sid * 8, _GPP - 8)

        t0 = _scan(cid * _NPASS, jnp.int32(0))

        def _pass(p, t_cur):
            p_global = cid * _NPASS + p
            parity = p & 1

            def _sa(i, c):
                pltpu.sync_copy(val2d.at[parity, i],
                                slab.at[pos2d.at[parity, i]], add=True)
                return c
            lax.fori_loop(0, t_cur, _sa, 0)
            plsc.subcore_barrier()

            gb = p_global * _GPP
            h = pltpu.async_copy(
                slab.at[pl.ds(gstart * 8192, 8 * 8192)],
                out_hbm.at[pl.ds((gb + gstart) * 8192, 8 * 8192)],
                dsem)
            t_next = _scan(p_global + 1, 1 - parity)
            h.wait()
            plsc.subcore_barrier()

            def _us(i, c):
                pltpu.sync_copy(zero128, slab.at[pos2d.at[parity, i]])
                return c
            lax.fori_loop(0, t_cur, _us, 0)
            plsc.subcore_barrier()
            return t_next

        lax.fori_loop(0, _NPASS, _pass, t0)

    return _scatter_kernel


def kernel(all_memory, last_memory, seq_item, Ur_w, Wr_w, Vr_w, Vr_b):
    del Vr_b
    probs = _probs_call(all_memory, last_memory, Ur_w.T, Wr_w.T, Vr_w)
    seq_pad = jnp.pad(seq_item, ((0, 0), (0, SPAD - SEQ_N)))
    flat = _get_scatter_kernel()(seq_pad, probs)
    out = flat.reshape(N_ITEMS // 8, 8, 8, 128).transpose((0, 2, 1, 3))
    return out.reshape(N_ITEMS, BATCH_N).T
